# Initial kernel scaffold; baseline (speedup 1.0000x reference)
import jax
import jax.numpy as jnp
from jax.experimental import pallas as pl


def kernel(indices, table):
    raise NotImplementedError("write your pallas kernel here")


# speedup vs baseline: 4.6774x; 4.6774x over previous
import functools

import jax
import jax.numpy as jnp
from jax import lax
from jax.experimental import pallas as pl
from jax.experimental.pallas import tpu as pltpu
from jax.experimental.pallas import tpu_sc as plsc

_VOCAB = 100000
_EMBED_DIM = 64
_BATCH = 4096
_HIST = 50

_NC = 2
_NS = 16
_NW = _NC * _NS

_TOTAL = _BATCH * _HIST
_PER_W = _TOTAL // _NW
_IW = 128
_ROWS_PER_W = _PER_W // _IW
_GROUP = 10
_OUTER = _ROWS_PER_W // _GROUP
_CHUNK = _GROUP * _IW


def _gather_body(table_hbm, idx_hbm, out_hbm, idx_v, rows_v, sem):
  wid = lax.axis_index("s") * _NC + lax.axis_index("c")
  base = wid * _PER_W
  pltpu.sync_copy(idx_hbm.at[wid], idx_v)

  def step(i, carry):
    copies = []
    for b in range(_GROUP):
      copies.append(
          pltpu.async_copy(
              table_hbm.at[idx_v.at[i * _GROUP + b]],
              rows_v.at[pl.ds(b * _IW, _IW)],
              sem,
          ))
    for c in copies:
      c.wait()
    pltpu.sync_copy(rows_v, out_hbm.at[pl.ds(base + i * _CHUNK, _CHUNK)])
    return carry

  lax.fori_loop(0, _OUTER, step, 0)


@functools.partial(jax.jit, static_argnames=())
def kernel(indices, table):
  idx = indices.reshape(-1).astype(jnp.int32).reshape(_NW, _ROWS_PER_W, _IW)
  mesh = plsc.VectorSubcoreMesh(core_axis_name="c", subcore_axis_name="s")
  out = pl.kernel(
      _gather_body,
      out_type=jax.ShapeDtypeStruct((_TOTAL, _EMBED_DIM), jnp.float32),
      mesh=mesh,
      scratch_types=[
          pltpu.VMEM((_ROWS_PER_W, _IW), jnp.int32),
          pltpu.VMEM((_CHUNK, _EMBED_DIM), jnp.float32),
          pltpu.SemaphoreType.DMA,
      ],
      compiler_params=pltpu.CompilerParams(use_tc_tiling_on_sc=False),
  )(table, idx)
  return out.reshape(_BATCH, _HIST, _EMBED_DIM)
